# Initial kernel scaffold; baseline (speedup 1.0000x reference)
import jax
import jax.numpy as jnp
from jax.experimental import pallas as pl


def kernel(num_queries, num_keys, weight):
    raise NotImplementedError("write your pallas kernel here")


# speedup vs baseline: 42.8312x; 42.8312x over previous
import functools

import jax
import jax.numpy as jnp
import numpy as np
from jax import lax
from jax.experimental import pallas as pl
from jax.experimental.pallas import tpu as pltpu
from jax.experimental.pallas import tpu_sc as plsc

NUM_BUCKETS = 32
MAX_DISTANCE = 128
NUM_HEADS = 16
SEQ = 2048
DIAGS = 2 * SEQ - 1
TPAD = 4480
ROWLEN = 4352
NSHIFT = 8
HEAD_WORDS = NSHIFT * ROWLEN
OUT_WORDS = NUM_HEADS * SEQ * SEQ

NC, NS = 2, 16


def _table_body(zero_ref, wt_ref, out_ref):
    d = lax.broadcasted_iota(jnp.int32, (NUM_HEADS, TPAD), 1)
    rel = d - (SEQ - 1) + zero_ref[0]
    nbh = NUM_BUCKETS // 2
    ret = jnp.where(rel >= 0, nbh, 0).astype(jnp.int32)
    n = jnp.abs(rel)
    max_exact = nbh // 2
    val_if_large = max_exact + (
        jnp.log(jnp.maximum(n, 1).astype(jnp.float32) / max_exact)
        / np.log(MAX_DISTANCE / max_exact)
        * (nbh - max_exact)
    ).astype(jnp.int32)
    val_if_large = jnp.minimum(val_if_large, nbh - 1)
    bucket = ret + jnp.where(n < max_exact, n, val_if_large)
    wt = wt_ref[...]
    acc = jnp.zeros((NUM_HEADS, TPAD), jnp.float32)
    for b in range(NUM_BUCKETS):
        acc = jnp.where(bucket == b, wt[:, b : b + 1], acc)
    for r in range(NSHIFT):
        out_ref[:, r, :] = acc[:, r : r + ROWLEN]


def _build_table(zero, weight):
    wt = weight.T
    return pl.pallas_call(
        _table_body,
        in_specs=[
            pl.BlockSpec(memory_space=pltpu.MemorySpace.SMEM),
            pl.BlockSpec(memory_space=pltpu.MemorySpace.VMEM),
        ],
        out_specs=pl.BlockSpec(memory_space=pltpu.MemorySpace.VMEM),
        out_shape=jax.ShapeDtypeStruct((NUM_HEADS, NSHIFT, ROWLEN), jnp.float32),
    )(zero, wt)


_ROWS_PER_WORKER = SEQ // NC
_CHUNK = 8


@functools.lru_cache(maxsize=1)
def _sc_broadcast_fn():
    mesh = plsc.VectorSubcoreMesh(
        core_axis_name="c", subcore_axis_name="s", num_cores=NC, num_subcores=NS
    )

    @functools.partial(
        pl.kernel,
        out_type=jax.ShapeDtypeStruct((OUT_WORDS,), jnp.float32),
        mesh=mesh,
        scratch_types=[
            pltpu.VMEM((HEAD_WORDS,), jnp.float32),
            pltpu.SemaphoreType.DMA,
        ],
    )
    def _sc_broadcast(t8_hbm, out_hbm, t8_v, sem):
        h = lax.axis_index("s")
        half = lax.axis_index("c")
        pltpu.sync_copy(t8_hbm.at[pl.ds(h * HEAD_WORDS, HEAD_WORDS)], t8_v)
        m0 = half * _ROWS_PER_WORKER
        out_head = h * (SEQ * SEQ)

        def body(i, carry):
            cps = []
            for j in range(_CHUNK):
                m = m0 + i * _CHUNK + j
                start = (SEQ - 1) - m
                r = lax.bitwise_and(start, NSHIFT - 1)
                src = pl.multiple_of(r * ROWLEN + (start - r), 8)
                dst = pl.multiple_of(out_head + m * SEQ, 8)
                cp = pltpu.make_async_copy(
                    t8_v.at[pl.ds(src, SEQ)],
                    out_hbm.at[pl.ds(dst, SEQ)],
                    sem,
                )
                cp.start()
                cps.append(cp)
            for cp in cps:
                cp.wait()
            return carry

        lax.fori_loop(0, _ROWS_PER_WORKER // _CHUNK, body, 0)

    return _sc_broadcast


def kernel(num_queries, num_keys, weight):
    zero = (jnp.asarray(num_queries, jnp.int32) - SEQ) + (
        jnp.asarray(num_keys, jnp.int32) - SEQ
    )
    t8 = _build_table(jnp.reshape(zero, (1,)), weight)
    flat = _sc_broadcast_fn()(jnp.reshape(t8, (-1,)))
    return jnp.reshape(flat, (1, NUM_HEADS, SEQ, SEQ))
